# speedup vs baseline: 1.1160x; 1.1160x over previous
import functools

import jax
import jax.numpy as jnp
from jax.experimental import pallas as pl
from jax.experimental.pallas import tpu as pltpu

DIM = 128
L = 20
B = 1024
SB = 8
SBL = SB * L
BB = 64
NSUB = BB // SB

_NEG = -9e15

_INTERPRET = False


def _leaky(x):
    return jnp.where(x >= 0, x, 0.2 * x)


def _tc_body(h_ref, adjb_ref, amat_ref, item_ref, maskf_ref,
             w1_ref, b1_ref, w2_ref, b2_ref, out_ref, anchor_ref):
    hflat = h_ref[...].reshape(BB * L, DIM)
    amat = amat_ref[...]
    for s in range(NSUB):
        hs = hflat[s * SBL:(s + 1) * SBL, :]
        u = jnp.concatenate(
            [hs * amat[k:k + 1, :] for k in range(4)], axis=0)
        e_all = jax.lax.dot_general(
            u, hs, (((1,), (1,)), ((), ())),
            preferred_element_type=jnp.float32)
        e_all = _leaky(e_all)
        adjb = adjb_ref[s]
        alpha = jnp.where(
            adjb == 2, e_all[0:SBL, :],
            jnp.where(adjb == 3, e_all[SBL:2 * SBL, :],
                      jnp.where(adjb == 4, e_all[2 * SBL:3 * SBL, :],
                                jnp.where(adjb == 5, e_all[3 * SBL:, :],
                                          jnp.where(adjb == 1, _NEG,
                                                    -jnp.inf)))))
        m = jnp.max(alpha, axis=1, keepdims=True)
        p = jnp.exp(alpha - m)
        p = p / jnp.sum(p, axis=1, keepdims=True)
        out_s = jax.lax.dot_general(
            p, hs, (((1,), (0,)), ((), ())),
            preferred_element_type=jnp.float32)
        out_ref[s * SB:(s + 1) * SB] = out_s.reshape(SB, L, DIM)

    maskf = maskf_ref[...]
    masked = item_ref[...] * maskf[:, :, None]
    s_emb = jnp.sum(masked, axis=1)
    cnt = jnp.sum(maskf, axis=1, keepdims=True)
    mean = s_emb / cnt
    hidden = jnp.maximum(
        jax.lax.dot_general(mean, w1_ref[...], (((1,), (0,)), ((), ())),
                            preferred_element_type=jnp.float32)
        + b1_ref[0:1, :], 0.0)
    anchor_ref[...] = (
        jax.lax.dot_general(hidden, w2_ref[...], (((1,), (0,)), ((), ())),
                            preferred_element_type=jnp.float32)
        + b2_ref[0:1, :])


def _tc_call(h, adjbig, amat, item_emb, maskf, w1, b1, w2, b2):
    grid = (B // BB,)
    return pl.pallas_call(
        _tc_body,
        grid=grid,
        in_specs=[
            pl.BlockSpec((BB, L, DIM), lambda i: (i, 0, 0)),
            pl.BlockSpec((NSUB, SBL, SBL), lambda i: (i, 0, 0)),
            pl.BlockSpec((8, DIM), lambda i: (0, 0)),
            pl.BlockSpec((BB, L, DIM), lambda i: (i, 0, 0)),
            pl.BlockSpec((BB, L), lambda i: (i, 0)),
            pl.BlockSpec((DIM, DIM), lambda i: (0, 0)),
            pl.BlockSpec((8, DIM), lambda i: (0, 0)),
            pl.BlockSpec((DIM, DIM), lambda i: (0, 0)),
            pl.BlockSpec((8, DIM), lambda i: (0, 0)),
        ],
        out_specs=[
            pl.BlockSpec((BB, L, DIM), lambda i: (i, 0, 0)),
            pl.BlockSpec((BB, DIM), lambda i: (i, 0)),
        ],
        out_shape=[
            jax.ShapeDtypeStruct((B, L, DIM), jnp.float32),
            jax.ShapeDtypeStruct((B, DIM), jnp.float32),
        ],
        interpret=_INTERPRET,
    )(h, adjbig, amat, item_emb, maskf, w1, b1, w2, b2)


def kernel(inputs, adj, mask_item, item, data, hg_adj, embedding, adj_all,
           num, a_0, a_1, a_2, a_3, mlp_w1, mlp_b1, mlp_w2, mlp_b2):
    h = jnp.take(embedding, inputs, axis=0)
    item_emb = jnp.take(embedding, item, axis=0)
    eye = jnp.eye(SB, dtype=jnp.int32)
    adjbig = ((adj.reshape(B // SB, SB, L, L) + 1)[:, :, :, None, :]
              * eye[None, :, None, :, None]).reshape(B // SB, SBL, SBL)
    amat = jnp.concatenate(
        [a_0.T, a_1.T, a_2.T, a_3.T,
         jnp.zeros((4, DIM), jnp.float32)], axis=0)
    maskf = mask_item.astype(jnp.float32)
    b1 = jnp.broadcast_to(mlp_b1[None, :], (8, DIM))
    b2 = jnp.broadcast_to(mlp_b2[None, :], (8, DIM))
    out, anchor = _tc_call(h, adjbig, amat, item_emb, maskf,
                           mlp_w1, b1, mlp_w2, b2)
    return (out, anchor)


# speedup vs baseline: 1.7010x; 1.5242x over previous
import functools

import jax
import jax.numpy as jnp
from jax import lax
from jax.experimental import pallas as pl
from jax.experimental.pallas import tpu as pltpu
from jax.experimental.pallas import tpu_sc as plsc

DIM = 128
L = 20
B = 1024
SB = 8
SBL = SB * L
BB = 64
NSUB = BB // SB

_NEG = -9e15

_INTERPRET = False

NC = 2
NS = 16
NW = NC * NS
TOT = 2 * B * L
PW = TOT // NW
CH = 128
NCH = PW // CH


def _sc_gather_body(table_hbm, idx_hbm, out_hbm,
                    idx_v, rows_a, rows_b, sem_a, sem_b):
    wid = lax.axis_index("s") * NC + lax.axis_index("c")
    base = wid * PW
    pltpu.sync_copy(idx_hbm.at[pl.ds(base, PW)], idx_v)

    bufs = (rows_a, rows_b)
    sems = (sem_a, sem_b)

    def start(c):
        pltpu.async_copy(
            table_hbm.at[idx_v.at[pl.ds(c * CH, CH)]], bufs[c % 2],
            sems[c % 2])

    start(0)
    for c in range(NCH):
        if c + 1 < NCH:
            start(c + 1)
        pltpu.make_async_copy(
            table_hbm.at[idx_v.at[pl.ds(c * CH, CH)]], bufs[c % 2],
            sems[c % 2]).wait()
        pltpu.sync_copy(bufs[c % 2], out_hbm.at[pl.ds(base + c * CH, CH)])


def _sc_gather(embedding, flat_idx):
    mesh = plsc.VectorSubcoreMesh(core_axis_name="c", subcore_axis_name="s")
    return pl.kernel(
        _sc_gather_body,
        mesh=mesh,
        out_type=jax.ShapeDtypeStruct((TOT, DIM), jnp.float32),
        scratch_types=[
            pltpu.VMEM((PW,), jnp.int32),
            pltpu.VMEM((CH, DIM), jnp.float32),
            pltpu.VMEM((CH, DIM), jnp.float32),
            pltpu.SemaphoreType.DMA,
            pltpu.SemaphoreType.DMA,
        ],
    )(embedding, flat_idx)


def _leaky(x):
    return jnp.where(x >= 0, x, 0.2 * x)


def _tc_body(h_ref, adjb_ref, amat_ref, item_ref, maskf_ref,
             w1_ref, b1_ref, w2_ref, b2_ref, out_ref, anchor_ref):
    hflat = h_ref[...]
    amat = amat_ref[...]
    for s in range(NSUB):
        hs = hflat[s * SBL:(s + 1) * SBL, :]
        u = jnp.concatenate(
            [hs * amat[k:k + 1, :] for k in range(4)], axis=0)
        e_all = jax.lax.dot_general(
            u, hs, (((1,), (1,)), ((), ())),
            preferred_element_type=jnp.float32)
        e_all = _leaky(e_all)
        adjb = adjb_ref[s].astype(jnp.int32)
        alpha = jnp.where(
            adjb == 2, e_all[0:SBL, :],
            jnp.where(adjb == 3, e_all[SBL:2 * SBL, :],
                      jnp.where(adjb == 4, e_all[2 * SBL:3 * SBL, :],
                                jnp.where(adjb == 5, e_all[3 * SBL:, :],
                                          jnp.where(adjb == 1, _NEG,
                                                    -jnp.inf)))))
        m = jnp.max(alpha, axis=1, keepdims=True)
        p = jnp.exp(alpha - m)
        p = p / jnp.sum(p, axis=1, keepdims=True)
        out_s = jax.lax.dot_general(
            p, hs, (((1,), (0,)), ((), ())),
            preferred_element_type=jnp.float32)
        out_ref[s * SB:(s + 1) * SB] = out_s.reshape(SB, L, DIM)

    maskf = maskf_ref[...]
    masked = item_ref[...].reshape(BB, L, DIM) * maskf[:, :, None]
    s_emb = jnp.sum(masked, axis=1)
    cnt = jnp.sum(maskf, axis=1, keepdims=True)
    mean = s_emb / cnt
    hidden = jnp.maximum(
        jax.lax.dot_general(mean, w1_ref[...], (((1,), (0,)), ((), ())),
                            preferred_element_type=jnp.float32)
        + b1_ref[0:1, :], 0.0)
    anchor_ref[...] = (
        jax.lax.dot_general(hidden, w2_ref[...], (((1,), (0,)), ((), ())),
                            preferred_element_type=jnp.float32)
        + b2_ref[0:1, :])


def _tc_call(gathered, adjbig, amat, maskf, w1, b1, w2, b2):
    grid = (B // BB,)
    nblk = B // BB
    return pl.pallas_call(
        _tc_body,
        grid=grid,
        in_specs=[
            pl.BlockSpec((BB * L, DIM), lambda i: (i, 0)),
            pl.BlockSpec((NSUB, SBL, SBL), lambda i: (i, 0, 0)),
            pl.BlockSpec((8, DIM), lambda i: (0, 0)),
            pl.BlockSpec((BB * L, DIM), lambda i, n=nblk: (i + n, 0)),
            pl.BlockSpec((BB, L), lambda i: (i, 0)),
            pl.BlockSpec((DIM, DIM), lambda i: (0, 0)),
            pl.BlockSpec((8, DIM), lambda i: (0, 0)),
            pl.BlockSpec((DIM, DIM), lambda i: (0, 0)),
            pl.BlockSpec((8, DIM), lambda i: (0, 0)),
        ],
        out_specs=[
            pl.BlockSpec((BB, L, DIM), lambda i: (i, 0, 0)),
            pl.BlockSpec((BB, DIM), lambda i: (i, 0)),
        ],
        out_shape=[
            jax.ShapeDtypeStruct((B, L, DIM), jnp.float32),
            jax.ShapeDtypeStruct((B, DIM), jnp.float32),
        ],
        interpret=_INTERPRET,
    )(gathered, adjbig, amat, gathered, maskf, w1, b1, w2, b2)


def kernel(inputs, adj, mask_item, item, data, hg_adj, embedding, adj_all,
           num, a_0, a_1, a_2, a_3, mlp_w1, mlp_b1, mlp_w2, mlp_b2):
    flat_idx = jnp.concatenate([inputs.reshape(-1), item.reshape(-1)])
    gathered = _sc_gather(embedding, flat_idx)
    eye = jnp.eye(SB, dtype=jnp.int8)
    adjbig = ((adj.reshape(B // SB, SB, L, L).astype(jnp.int8) + 1)
              [:, :, :, None, :]
              * eye[None, :, None, :, None]).reshape(B // SB, SBL, SBL)
    amat = jnp.concatenate(
        [a_0.T, a_1.T, a_2.T, a_3.T,
         jnp.zeros((4, DIM), jnp.float32)], axis=0)
    maskf = mask_item.astype(jnp.float32)
    b1 = jnp.broadcast_to(mlp_b1[None, :], (8, DIM))
    b2 = jnp.broadcast_to(mlp_b2[None, :], (8, DIM))
    out, anchor = _tc_call(gathered, adjbig, amat, maskf,
                           mlp_w1, b1, mlp_w2, b2)
    return (out, anchor)


# speedup vs baseline: 1.9883x; 1.1689x over previous
import functools

import jax
import jax.numpy as jnp
from jax import lax
from jax.experimental import pallas as pl
from jax.experimental.pallas import tpu as pltpu
from jax.experimental.pallas import tpu_sc as plsc

DIM = 128
L = 20
B = 1024
SB = 8
SBL = SB * L
BB = 64
NSUB = BB // SB

_NEG = -9e15

_INTERPRET = False

NC = 2
NS = 16
NW = NC * NS
TOT = 2 * B * L
PW = TOT // NW
CH = 128
NCH = PW // CH


def _sc_gather_body(table_hbm, idx_hbm, out_hbm,
                    idx_v, rows_a, rows_b, sem_a, sem_b):
    wid = lax.axis_index("s") * NC + lax.axis_index("c")
    base = wid * PW
    pltpu.sync_copy(idx_hbm.at[pl.ds(base, PW)], idx_v)

    bufs = (rows_a, rows_b)
    sems = (sem_a, sem_b)

    def start(c):
        pltpu.async_copy(
            table_hbm.at[idx_v.at[pl.ds(c * CH, CH)]], bufs[c % 2],
            sems[c % 2])

    start(0)
    for c in range(NCH):
        if c + 1 < NCH:
            start(c + 1)
        pltpu.make_async_copy(
            table_hbm.at[idx_v.at[pl.ds(c * CH, CH)]], bufs[c % 2],
            sems[c % 2]).wait()
        pltpu.sync_copy(bufs[c % 2], out_hbm.at[pl.ds(base + c * CH, CH)])


def _sc_gather(embedding, flat_idx):
    mesh = plsc.VectorSubcoreMesh(core_axis_name="c", subcore_axis_name="s")
    return pl.kernel(
        _sc_gather_body,
        mesh=mesh,
        out_type=jax.ShapeDtypeStruct((TOT, DIM), jnp.float32),
        scratch_types=[
            pltpu.VMEM((PW,), jnp.int32),
            pltpu.VMEM((CH, DIM), jnp.float32),
            pltpu.VMEM((CH, DIM), jnp.float32),
            pltpu.SemaphoreType.DMA,
            pltpu.SemaphoreType.DMA,
        ],
    )(embedding, flat_idx)


def _leaky(x):
    return jnp.where(x >= 0, x, 0.2 * x)


def _tc_body(h_ref, adj_ref, amat_ref, item_ref, maskf_ref,
             w1_ref, b1_ref, w2_ref, b2_ref, out_ref, anchor_ref):
    hflat = h_ref[...]
    amat = amat_ref[...]
    adjflat = adj_ref[...].reshape(BB * L, L)
    rg = jax.lax.broadcasted_iota(jnp.int32, (SBL, SBL), 0) // L
    cg = jax.lax.broadcasted_iota(jnp.int32, (SBL, SBL), 1) // L
    bd = rg == cg
    for s in range(NSUB):
        hs = hflat[s * SBL:(s + 1) * SBL, :]
        u = jnp.concatenate(
            [hs * amat[k:k + 1, :] for k in range(4)], axis=0)
        e_all = jax.lax.dot_general(
            u, hs, (((1,), (1,)), ((), ())),
            preferred_element_type=jnp.float32)
        adjs = adjflat[s * SBL:(s + 1) * SBL, :]
        rep = jnp.concatenate([adjs] * SB, axis=1)
        mr = jnp.where(bd, rep, 5)
        alpha = jnp.where(
            mr == 1, e_all[0:SBL, :],
            jnp.where(mr == 2, e_all[SBL:2 * SBL, :],
                      jnp.where(mr == 3, e_all[2 * SBL:3 * SBL, :],
                                jnp.where(mr == 4, e_all[3 * SBL:, :],
                                          jnp.where(mr == 5, -jnp.inf,
                                                    _NEG)))))
        alpha = _leaky(alpha)
        m = jnp.max(alpha, axis=1, keepdims=True)
        p = jnp.exp(alpha - m)
        p = p / jnp.sum(p, axis=1, keepdims=True)
        out_s = jax.lax.dot_general(
            p, hs, (((1,), (0,)), ((), ())),
            preferred_element_type=jnp.float32)
        out_ref[s * SB:(s + 1) * SB] = out_s.reshape(SB, L, DIM)

    maskf = maskf_ref[...].astype(jnp.float32)
    masked = item_ref[...].reshape(BB, L, DIM) * maskf[:, :, None]
    s_emb = jnp.sum(masked, axis=1)
    cnt = jnp.sum(maskf, axis=1, keepdims=True)
    mean = s_emb / cnt
    hidden = jnp.maximum(
        jax.lax.dot_general(mean, w1_ref[...], (((1,), (0,)), ((), ())),
                            preferred_element_type=jnp.float32)
        + b1_ref[0:1, :], 0.0)
    anchor_ref[...] = (
        jax.lax.dot_general(hidden, w2_ref[...], (((1,), (0,)), ((), ())),
                            preferred_element_type=jnp.float32)
        + b2_ref[0:1, :])


def _tc_call(gathered, adj, amat, mask_item, w1, b1, w2, b2):
    grid = (B // BB,)
    nblk = B // BB
    return pl.pallas_call(
        _tc_body,
        grid=grid,
        in_specs=[
            pl.BlockSpec((BB * L, DIM), lambda i: (i, 0)),
            pl.BlockSpec((BB, L, L), lambda i: (i, 0, 0)),
            pl.BlockSpec((8, DIM), lambda i: (0, 0)),
            pl.BlockSpec((BB * L, DIM), lambda i, n=nblk: (i + n, 0)),
            pl.BlockSpec((BB, L), lambda i: (i, 0)),
            pl.BlockSpec((DIM, DIM), lambda i: (0, 0)),
            pl.BlockSpec((8, DIM), lambda i: (0, 0)),
            pl.BlockSpec((DIM, DIM), lambda i: (0, 0)),
            pl.BlockSpec((8, DIM), lambda i: (0, 0)),
        ],
        out_specs=[
            pl.BlockSpec((BB, L, DIM), lambda i: (i, 0, 0)),
            pl.BlockSpec((BB, DIM), lambda i: (i, 0)),
        ],
        out_shape=[
            jax.ShapeDtypeStruct((B, L, DIM), jnp.float32),
            jax.ShapeDtypeStruct((B, DIM), jnp.float32),
        ],
        interpret=_INTERPRET,
    )(gathered, adj, amat, gathered, mask_item, w1, b1, w2, b2)


def kernel(inputs, adj, mask_item, item, data, hg_adj, embedding, adj_all,
           num, a_0, a_1, a_2, a_3, mlp_w1, mlp_b1, mlp_w2, mlp_b2):
    flat_idx = jnp.concatenate([inputs.reshape(-1), item.reshape(-1)])
    gathered = _sc_gather(embedding, flat_idx)
    amat = jnp.concatenate(
        [a_0.T, a_1.T, a_2.T, a_3.T,
         jnp.zeros((4, DIM), jnp.float32)], axis=0)
    b1 = jnp.broadcast_to(mlp_b1[None, :], (8, DIM))
    b2 = jnp.broadcast_to(mlp_b2[None, :], (8, DIM))
    out, anchor = _tc_call(gathered, adj, amat, mask_item,
                           mlp_w1, b1, mlp_w2, b2)
    return (out, anchor)


# speedup vs baseline: 2.5000x; 1.2574x over previous
import functools

import jax
import jax.numpy as jnp
from jax import lax
from jax.experimental import pallas as pl
from jax.experimental.pallas import tpu as pltpu
from jax.experimental.pallas import tpu_sc as plsc

DIM = 128
L = 20
B = 1024
SB = 8
SBL = SB * L
BB = 64
NSUB = BB // SB

_NEG = -9e15

_INTERPRET = False

NC = 2
NS = 16
NW = NC * NS
TOT = 2 * B * L
PW = TOT // NW
CH = 128
NCH = PW // CH


def _sc_gather_body(table_hbm, idx_hbm, out_hbm,
                    idx_v, rows_a, rows_b, sem_a, sem_b):
    wid = lax.axis_index("s") * NC + lax.axis_index("c")
    base = wid * PW
    pltpu.sync_copy(idx_hbm.at[pl.ds(base, PW)], idx_v)

    bufs = (rows_a, rows_b)
    sems = (sem_a, sem_b)

    def start(c):
        pltpu.async_copy(
            table_hbm.at[idx_v.at[pl.ds(c * CH, CH)]], bufs[c % 2],
            sems[c % 2])

    start(0)
    for c in range(NCH):
        if c + 1 < NCH:
            start(c + 1)
        pltpu.make_async_copy(
            table_hbm.at[idx_v.at[pl.ds(c * CH, CH)]], bufs[c % 2],
            sems[c % 2]).wait()
        pltpu.sync_copy(bufs[c % 2], out_hbm.at[pl.ds(base + c * CH, CH)])


def _sc_gather(embedding, flat_idx):
    mesh = plsc.VectorSubcoreMesh(core_axis_name="c", subcore_axis_name="s")
    return pl.kernel(
        _sc_gather_body,
        mesh=mesh,
        out_type=jax.ShapeDtypeStruct((TOT, DIM), jnp.float32),
        scratch_types=[
            pltpu.VMEM((PW,), jnp.int32),
            pltpu.VMEM((CH, DIM), jnp.float32),
            pltpu.VMEM((CH, DIM), jnp.float32),
            pltpu.SemaphoreType.DMA,
            pltpu.SemaphoreType.DMA,
        ],
    )(embedding, flat_idx)


def _leaky(x):
    return jnp.where(x >= 0, x, 0.2 * x)


def _tc_body(h_ref, adj_ref, amat_ref, item_ref, maskf_ref,
             w1_ref, b1_ref, w2_ref, b2_ref, out_ref, anchor_ref):
    hflat = h_ref[...]
    amat = amat_ref[...]
    rg = jax.lax.broadcasted_iota(jnp.int32, (SBL, SBL), 0) // L
    cg = jax.lax.broadcasted_iota(jnp.int32, (SBL, SBL), 1) // L
    bd = rg == cg
    selc = jax.lax.broadcasted_iota(jnp.int32, (L, SBL), 0)
    selj = jax.lax.broadcasted_iota(jnp.int32, (L, SBL), 1) % L
    sel = (selc == selj).astype(jnp.float32)

    hss, e_alls, mrs = [], [], []
    for s in range(NSUB):
        hs = hflat[s * SBL:(s + 1) * SBL, :]
        u = jnp.concatenate(
            [hs * amat[k:k + 1, :] for k in range(4)], axis=0)
        e_all = jax.lax.dot_general(
            u, hs, (((1,), (1,)), ((), ())),
            preferred_element_type=jnp.float32)
        rep = jax.lax.dot_general(
            adj_ref[s].astype(jnp.float32), sel, (((1,), (0,)), ((), ())),
            preferred_element_type=jnp.float32)
        hss.append(hs)
        e_alls.append(e_all)
        mrs.append(jnp.where(bd, rep, 5.0))

    for s in range(NSUB):
        e_all, mr = e_alls[s], mrs[s]
        alpha = jnp.where(
            mr == 1.0, e_all[0:SBL, :],
            jnp.where(mr == 2.0, e_all[SBL:2 * SBL, :],
                      jnp.where(mr == 3.0, e_all[2 * SBL:3 * SBL, :],
                                jnp.where(mr == 4.0, e_all[3 * SBL:, :],
                                          jnp.where(mr == 5.0, -jnp.inf,
                                                    _NEG)))))
        alpha = _leaky(alpha)
        m = jnp.max(alpha, axis=1, keepdims=True)
        p = jnp.exp(alpha - m)
        p = p / jnp.sum(p, axis=1, keepdims=True)
        out_s = jax.lax.dot_general(
            p, hss[s], (((1,), (0,)), ((), ())),
            preferred_element_type=jnp.float32)
        out_ref[s * SB:(s + 1) * SB] = out_s.reshape(SB, L, DIM)

    maskf = maskf_ref[...].astype(jnp.float32)
    masked = item_ref[...].reshape(BB, L, DIM) * maskf[:, :, None]
    s_emb = jnp.sum(masked, axis=1)
    cnt = jnp.sum(maskf, axis=1, keepdims=True)
    mean = s_emb / cnt
    hidden = jnp.maximum(
        jax.lax.dot_general(mean, w1_ref[...], (((1,), (0,)), ((), ())),
                            preferred_element_type=jnp.float32)
        + b1_ref[0:1, :], 0.0)
    anchor_ref[...] = (
        jax.lax.dot_general(hidden, w2_ref[...], (((1,), (0,)), ((), ())),
                            preferred_element_type=jnp.float32)
        + b2_ref[0:1, :])


def _tc_call(gathered, adj, amat, mask_item, w1, b1, w2, b2):
    grid = (B // BB,)
    nblk = B // BB
    return pl.pallas_call(
        _tc_body,
        grid=grid,
        in_specs=[
            pl.BlockSpec((BB * L, DIM), lambda i: (i, 0)),
            pl.BlockSpec((NSUB, SBL, L), lambda i: (i, 0, 0)),
            pl.BlockSpec((8, DIM), lambda i: (0, 0)),
            pl.BlockSpec((BB * L, DIM), lambda i, n=nblk: (i + n, 0)),
            pl.BlockSpec((BB, L), lambda i: (i, 0)),
            pl.BlockSpec((DIM, DIM), lambda i: (0, 0)),
            pl.BlockSpec((8, DIM), lambda i: (0, 0)),
            pl.BlockSpec((DIM, DIM), lambda i: (0, 0)),
            pl.BlockSpec((8, DIM), lambda i: (0, 0)),
        ],
        out_specs=[
            pl.BlockSpec((BB, L, DIM), lambda i: (i, 0, 0)),
            pl.BlockSpec((BB, DIM), lambda i: (i, 0)),
        ],
        out_shape=[
            jax.ShapeDtypeStruct((B, L, DIM), jnp.float32),
            jax.ShapeDtypeStruct((B, DIM), jnp.float32),
        ],
        interpret=_INTERPRET,
    )(gathered, adj, amat, gathered, mask_item, w1, b1, w2, b2)


def kernel(inputs, adj, mask_item, item, data, hg_adj, embedding, adj_all,
           num, a_0, a_1, a_2, a_3, mlp_w1, mlp_b1, mlp_w2, mlp_b2):
    flat_idx = jnp.concatenate([inputs.reshape(-1), item.reshape(-1)])
    gathered = _sc_gather(embedding, flat_idx)
    adjr = adj.reshape(B * L // SBL, SBL, L)
    amat = jnp.concatenate(
        [a_0.T, a_1.T, a_2.T, a_3.T,
         jnp.zeros((4, DIM), jnp.float32)], axis=0)
    b1 = jnp.broadcast_to(mlp_b1[None, :], (8, DIM))
    b2 = jnp.broadcast_to(mlp_b2[None, :], (8, DIM))
    out, anchor = _tc_call(gathered, adjr, amat, mask_item,
                           mlp_w1, b1, mlp_w2, b2)
    return (out, anchor)


# speedup vs baseline: 3.1100x; 1.2440x over previous
import functools

import jax
import jax.numpy as jnp
from jax import lax
from jax.experimental import pallas as pl
from jax.experimental.pallas import tpu as pltpu
from jax.experimental.pallas import tpu_sc as plsc

DIM = 128
L = 20
B = 1024
SB = 8
SBL = SB * L
BB = 64
NSUB = BB // SB

_NEG = -9e15

_INTERPRET = False

NC = 2
NS = 16
NW = NC * NS
TOT = 2 * B * L
PW = TOT // NW
CH = 128
NCH = PW // CH


PWH = (B * L) // NW
NCHH = PWH // CH


def _sc_gather_body(table_hbm, idx_hbm, out_h, out_i,
                    idx_v, rows_a, rows_b, sem_a, sem_b):
    wid = lax.axis_index("s") * NC + lax.axis_index("c")
    base = wid * PWH
    pltpu.sync_copy(idx_hbm.at[pl.ds(base, PWH)], idx_v.at[pl.ds(0, PWH)])
    pltpu.sync_copy(idx_hbm.at[pl.ds(B * L + base, PWH)],
                    idx_v.at[pl.ds(PWH, PWH)])

    bufs = (rows_a, rows_b)
    sems = (sem_a, sem_b)

    def start(c):
        pltpu.async_copy(
            table_hbm.at[idx_v.at[pl.ds(c * CH, CH)]], bufs[c % 2],
            sems[c % 2])

    start(0)
    for c in range(NCH):
        if c + 1 < NCH:
            start(c + 1)
        pltpu.make_async_copy(
            table_hbm.at[idx_v.at[pl.ds(c * CH, CH)]], bufs[c % 2],
            sems[c % 2]).wait()
        if c < NCHH:
            dst = out_h.at[pl.ds(base + c * CH, CH)]
        else:
            dst = out_i.at[pl.ds(base + (c - NCHH) * CH, CH)]
        pltpu.sync_copy(bufs[c % 2], dst)


def _sc_gather(embedding, flat_idx):
    mesh = plsc.VectorSubcoreMesh(core_axis_name="c", subcore_axis_name="s")
    return pl.kernel(
        _sc_gather_body,
        mesh=mesh,
        out_type=[
            jax.ShapeDtypeStruct((B * L, DIM), jnp.float32),
            jax.ShapeDtypeStruct((B * L, DIM), jnp.float32),
        ],
        scratch_types=[
            pltpu.VMEM((PW,), jnp.int32),
            pltpu.VMEM((CH, DIM), jnp.float32),
            pltpu.VMEM((CH, DIM), jnp.float32),
            pltpu.SemaphoreType.DMA,
            pltpu.SemaphoreType.DMA,
        ],
    )(embedding, flat_idx)


def _leaky(x):
    return jnp.where(x >= 0, x, 0.2 * x)


def _tc_body(h_ref, adj_ref, amat_ref, item_ref, maskf_ref,
             w1_ref, b1_ref, w2_ref, b2_ref, out_ref, anchor_ref):
    hload = h_ref[...]
    amat = amat_ref[...]
    adjload = adj_ref[...]
    rg = jax.lax.broadcasted_iota(jnp.int32, (SBL, SBL), 0) % SB
    cg = jax.lax.broadcasted_iota(jnp.int32, (SBL, SBL), 1) % SB
    bd = rg == cg
    selc = jax.lax.broadcasted_iota(jnp.int32, (L, SBL), 0)
    selj = jax.lax.broadcasted_iota(jnp.int32, (L, SBL), 1) // SB
    sel = (selc == selj).astype(jnp.float32)

    hss, e_alls, mrs = [], [], []
    for s in range(NSUB):
        hs = hload[:, s * SB:(s + 1) * SB, :].reshape(SBL, DIM)
        u = jnp.concatenate(
            [hs * amat[k:k + 1, :] for k in range(4)], axis=0)
        e_all = jax.lax.dot_general(
            u, hs, (((1,), (1,)), ((), ())),
            preferred_element_type=jnp.float32)
        adjs = adjload[:, s * SB:(s + 1) * SB, :].reshape(SBL, L)
        rep = jax.lax.dot_general(
            adjs.astype(jnp.float32), sel, (((1,), (0,)), ((), ())),
            preferred_element_type=jnp.float32)
        hss.append(hs)
        e_alls.append(e_all)
        mrs.append(jnp.where(bd, rep, 5.0))

    for s in range(NSUB):
        e_all, mr = e_alls[s], mrs[s]
        alpha = jnp.where(
            mr == 1.0, e_all[0:SBL, :],
            jnp.where(mr == 2.0, e_all[SBL:2 * SBL, :],
                      jnp.where(mr == 3.0, e_all[2 * SBL:3 * SBL, :],
                                jnp.where(mr == 4.0, e_all[3 * SBL:, :],
                                          jnp.where(mr == 5.0, -jnp.inf,
                                                    _NEG)))))
        alpha = _leaky(alpha)
        m = jnp.max(alpha, axis=1, keepdims=True)
        p = jnp.exp(alpha - m)
        p = p / jnp.sum(p, axis=1, keepdims=True)
        out_s = jax.lax.dot_general(
            p, hss[s], (((1,), (0,)), ((), ())),
            preferred_element_type=jnp.float32)
        out_ref[:, s * SB:(s + 1) * SB, :] = out_s.reshape(L, SB, DIM)

    maskf = maskf_ref[...].astype(jnp.float32)
    masked = item_ref[...].reshape(BB, L, DIM) * maskf[:, :, None]
    s_emb = jnp.sum(masked, axis=1)
    cnt = jnp.sum(maskf, axis=1, keepdims=True)
    mean = s_emb / cnt
    hidden = jnp.maximum(
        jax.lax.dot_general(mean, w1_ref[...], (((1,), (0,)), ((), ())),
                            preferred_element_type=jnp.float32)
        + b1_ref[0:1, :], 0.0)
    anchor_ref[...] = (
        jax.lax.dot_general(hidden, w2_ref[...], (((1,), (0,)), ((), ())),
                            preferred_element_type=jnp.float32)
        + b2_ref[0:1, :])


def _tc_call(ht, item_rows, adj_t, amat, mask_item, w1, b1, w2, b2):
    grid = (B // BB,)
    return pl.pallas_call(
        _tc_body,
        grid=grid,
        in_specs=[
            pl.BlockSpec((L, BB, DIM), lambda i: (0, i, 0)),
            pl.BlockSpec((L, BB, L), lambda i: (0, i, 0)),
            pl.BlockSpec((8, DIM), lambda i: (0, 0)),
            pl.BlockSpec((BB * L, DIM), lambda i: (i, 0)),
            pl.BlockSpec((BB, L), lambda i: (i, 0)),
            pl.BlockSpec((DIM, DIM), lambda i: (0, 0)),
            pl.BlockSpec((8, DIM), lambda i: (0, 0)),
            pl.BlockSpec((DIM, DIM), lambda i: (0, 0)),
            pl.BlockSpec((8, DIM), lambda i: (0, 0)),
        ],
        out_specs=[
            pl.BlockSpec((L, BB, DIM), lambda i: (0, i, 0)),
            pl.BlockSpec((BB, DIM), lambda i: (i, 0)),
        ],
        out_shape=[
            jax.ShapeDtypeStruct((L, B, DIM), jnp.float32),
            jax.ShapeDtypeStruct((B, DIM), jnp.float32),
        ],
        interpret=_INTERPRET,
    )(ht, adj_t, amat, item_rows, mask_item, w1, b1, w2, b2)


def kernel(inputs, adj, mask_item, item, data, hg_adj, embedding, adj_all,
           num, a_0, a_1, a_2, a_3, mlp_w1, mlp_b1, mlp_w2, mlp_b2):
    flat_idx = jnp.concatenate([inputs.T.reshape(-1), item.reshape(-1)])
    h_flat, item_rows = _sc_gather(embedding, flat_idx)
    ht = h_flat.reshape(L, B, DIM)
    adj_t = adj.transpose(1, 0, 2)
    amat = jnp.concatenate(
        [a_0.T, a_1.T, a_2.T, a_3.T,
         jnp.zeros((4, DIM), jnp.float32)], axis=0)
    b1 = jnp.broadcast_to(mlp_b1[None, :], (8, DIM))
    b2 = jnp.broadcast_to(mlp_b2[None, :], (8, DIM))
    out_t, anchor = _tc_call(ht, item_rows, adj_t, amat, mask_item,
                             mlp_w1, b1, mlp_w2, b2)
    return (out_t.transpose(1, 0, 2), anchor)


# speedup vs baseline: 3.1868x; 1.0247x over previous
import functools

import jax
import jax.numpy as jnp
from jax import lax
from jax.experimental import pallas as pl
from jax.experimental.pallas import tpu as pltpu
from jax.experimental.pallas import tpu_sc as plsc

DIM = 128
L = 20
B = 1024
SB = 8
SBL = SB * L
BB = 64
NSUB = BB // SB

_NEG = -9e15

_INTERPRET = False

NC = 2
NS = 16
NW = NC * NS
TOT = 2 * B * L
PW = TOT // NW
CH = 128
NCH = PW // CH


PWH = (B * L) // NW
NCHH = PWH // CH
SPW = B // NW


def _sc_gather_body(table_hbm, idx_hbm, mask_hbm, out_h, out_s,
                    idx_v, rows_a, rows_b, big_i, mask_v, sums_v,
                    sem_a, sem_b, sem_i):
    wid = lax.axis_index("s") * NC + lax.axis_index("c")
    base = wid * PWH
    pltpu.sync_copy(idx_hbm.at[pl.ds(base, PWH)], idx_v.at[pl.ds(0, PWH)])
    pltpu.sync_copy(idx_hbm.at[pl.ds(B * L + base, PWH)],
                    idx_v.at[pl.ds(PWH, PWH)])
    pltpu.sync_copy(mask_hbm.at[pl.ds(wid * SPW, SPW)], mask_v)

    def item_copy(c):
        return pltpu.make_async_copy(
            table_hbm.at[idx_v.at[pl.ds(PWH + c * CH, CH)]],
            big_i.at[pl.ds(c * CH, CH)], sem_i)

    for c in range(NCHH):
        pltpu.async_copy(
            table_hbm.at[idx_v.at[pl.ds(PWH + c * CH, CH)]],
            big_i.at[pl.ds(c * CH, CH)], sem_i)

    bufs = (rows_a, rows_b)
    sems = (sem_a, sem_b)

    def start(c):
        pltpu.async_copy(
            table_hbm.at[idx_v.at[pl.ds(c * CH, CH)]], bufs[c % 2],
            sems[c % 2])

    start(0)
    for c in range(NCHH):
        if c + 1 < NCHH:
            start(c + 1)
        pltpu.make_async_copy(
            table_hbm.at[idx_v.at[pl.ds(c * CH, CH)]], bufs[c % 2],
            sems[c % 2]).wait()
        pltpu.sync_copy(bufs[c % 2], out_h.at[pl.ds(base + c * CH, CH)])

    for c in range(NCHH):
        item_copy(c).wait()

    def session(j, _):
        ma = mask_v[j, pl.ds(0, 16)]
        mb = mask_v[j, pl.ds(16, 16)]
        accs = [jnp.zeros((16,), jnp.float32) for _ in range(DIM // 16)]
        for r in range(L):
            mv = ma[r] if r < 16 else mb[r - 16]
            for cc in range(DIM // 16):
                accs[cc] = accs[cc] + mv * big_i[j * L + r,
                                                 pl.ds(cc * 16, 16)]
        for cc in range(DIM // 16):
            sums_v[j, pl.ds(cc * 16, 16)] = accs[cc]
        return 0

    jax.lax.fori_loop(0, SPW, session, 0)
    pltpu.sync_copy(sums_v, out_s.at[pl.ds(wid * SPW, SPW)])


def _sc_gather(embedding, flat_idx, mask_item):
    mesh = plsc.VectorSubcoreMesh(core_axis_name="c", subcore_axis_name="s")
    return pl.kernel(
        _sc_gather_body,
        mesh=mesh,
        out_type=[
            jax.ShapeDtypeStruct((B * L, DIM), jnp.float32),
            jax.ShapeDtypeStruct((B, DIM), jnp.float32),
        ],
        scratch_types=[
            pltpu.VMEM((PW,), jnp.int32),
            pltpu.VMEM((CH, DIM), jnp.float32),
            pltpu.VMEM((CH, DIM), jnp.float32),
            pltpu.VMEM((PWH, DIM), jnp.float32),
            pltpu.VMEM((SPW, 32), jnp.float32),
            pltpu.VMEM((SPW, DIM), jnp.float32),
            pltpu.SemaphoreType.DMA,
            pltpu.SemaphoreType.DMA,
            pltpu.SemaphoreType.DMA,
        ],
    )(embedding, flat_idx, mask_item)


def _leaky(x):
    return jnp.where(x >= 0, x, 0.2 * x)


def _tc_body(h_ref, adj_ref, amat_ref, sums_ref, maskf_ref,
             w1_ref, b1_ref, w2_ref, b2_ref, out_ref, anchor_ref):
    hload = h_ref[...]
    amat = amat_ref[...]
    adjload = adj_ref[...]
    rg = jax.lax.broadcasted_iota(jnp.int32, (SBL, SBL), 0) % SB
    cg = jax.lax.broadcasted_iota(jnp.int32, (SBL, SBL), 1) % SB
    bd = rg == cg
    selc = jax.lax.broadcasted_iota(jnp.int32, (L, SBL), 0)
    selj = jax.lax.broadcasted_iota(jnp.int32, (L, SBL), 1) // SB
    sel = (selc == selj).astype(jnp.float32)

    hss, e_alls, mrs = [], [], []
    for s in range(NSUB):
        hs = hload[:, s * SB:(s + 1) * SB, :].reshape(SBL, DIM)
        u = jnp.concatenate(
            [hs * amat[k:k + 1, :] for k in range(4)], axis=0)
        e_all = jax.lax.dot_general(
            u, hs, (((1,), (1,)), ((), ())),
            preferred_element_type=jnp.float32)
        adjs = adjload[:, s * SB:(s + 1) * SB, :].reshape(SBL, L)
        rep = jax.lax.dot_general(
            adjs.astype(jnp.float32), sel, (((1,), (0,)), ((), ())),
            preferred_element_type=jnp.float32)
        hss.append(hs)
        e_alls.append(e_all)
        mrs.append(jnp.where(bd, rep, 5.0))

    for s in range(NSUB):
        e_all, mr = e_alls[s], mrs[s]
        alpha = jnp.where(
            mr == 1.0, e_all[0:SBL, :],
            jnp.where(mr == 2.0, e_all[SBL:2 * SBL, :],
                      jnp.where(mr == 3.0, e_all[2 * SBL:3 * SBL, :],
                                jnp.where(mr == 4.0, e_all[3 * SBL:, :],
                                          jnp.where(mr == 5.0, -jnp.inf,
                                                    _NEG)))))
        alpha = _leaky(alpha)
        m = jnp.max(alpha, axis=1, keepdims=True)
        p = jnp.exp(alpha - m)
        p = p / jnp.sum(p, axis=1, keepdims=True)
        out_s = jax.lax.dot_general(
            p, hss[s], (((1,), (0,)), ((), ())),
            preferred_element_type=jnp.float32)
        out_ref[:, s * SB:(s + 1) * SB, :] = out_s.reshape(L, SB, DIM)

    maskf = maskf_ref[...].astype(jnp.float32)
    s_emb = sums_ref[...]
    cnt = jnp.sum(maskf, axis=1, keepdims=True)
    mean = s_emb / cnt
    hidden = jnp.maximum(
        jax.lax.dot_general(mean, w1_ref[...], (((1,), (0,)), ((), ())),
                            preferred_element_type=jnp.float32)
        + b1_ref[0:1, :], 0.0)
    anchor_ref[...] = (
        jax.lax.dot_general(hidden, w2_ref[...], (((1,), (0,)), ((), ())),
                            preferred_element_type=jnp.float32)
        + b2_ref[0:1, :])


def _tc_call(ht, item_sums, adj_t, amat, mask_item, w1, b1, w2, b2):
    grid = (B // BB,)
    return pl.pallas_call(
        _tc_body,
        grid=grid,
        in_specs=[
            pl.BlockSpec((L, BB, DIM), lambda i: (0, i, 0)),
            pl.BlockSpec((L, BB, L), lambda i: (0, i, 0)),
            pl.BlockSpec((8, DIM), lambda i: (0, 0)),
            pl.BlockSpec((BB, DIM), lambda i: (i, 0)),
            pl.BlockSpec((BB, L), lambda i: (i, 0)),
            pl.BlockSpec((DIM, DIM), lambda i: (0, 0)),
            pl.BlockSpec((8, DIM), lambda i: (0, 0)),
            pl.BlockSpec((DIM, DIM), lambda i: (0, 0)),
            pl.BlockSpec((8, DIM), lambda i: (0, 0)),
        ],
        out_specs=[
            pl.BlockSpec((L, BB, DIM), lambda i: (0, i, 0)),
            pl.BlockSpec((BB, DIM), lambda i: (i, 0)),
        ],
        out_shape=[
            jax.ShapeDtypeStruct((L, B, DIM), jnp.float32),
            jax.ShapeDtypeStruct((B, DIM), jnp.float32),
        ],
        interpret=_INTERPRET,
    )(ht, adj_t, amat, item_sums, mask_item, w1, b1, w2, b2)


def kernel(inputs, adj, mask_item, item, data, hg_adj, embedding, adj_all,
           num, a_0, a_1, a_2, a_3, mlp_w1, mlp_b1, mlp_w2, mlp_b2):
    flat_idx = jnp.concatenate([inputs.T.reshape(-1), item.reshape(-1)])
    maskp = jnp.pad(mask_item.astype(jnp.float32), ((0, 0), (0, 32 - L)))
    h_flat, item_sums = _sc_gather(embedding, flat_idx, maskp)
    ht = h_flat.reshape(L, B, DIM)
    adj_t = adj.transpose(1, 0, 2)
    amat = jnp.concatenate(
        [a_0.T, a_1.T, a_2.T, a_3.T,
         jnp.zeros((4, DIM), jnp.float32)], axis=0)
    b1 = jnp.broadcast_to(mlp_b1[None, :], (8, DIM))
    b2 = jnp.broadcast_to(mlp_b2[None, :], (8, DIM))
    out_t, anchor = _tc_call(ht, item_sums, adj_t, amat, mask_item,
                             mlp_w1, b1, mlp_w2, b2)
    return (out_t.transpose(1, 0, 2), anchor)


# speedup vs baseline: 3.3127x; 1.0395x over previous
import functools

import jax
import jax.numpy as jnp
from jax import lax
from jax.experimental import pallas as pl
from jax.experimental.pallas import tpu as pltpu
from jax.experimental.pallas import tpu_sc as plsc

DIM = 128
L = 20
B = 1024
SB = 8
SBL = SB * L
BB = 128
NSUB = BB // SB

_NEG = -9e15

_INTERPRET = False

NC = 2
NS = 16
NW = NC * NS
TOT = 2 * B * L
PW = TOT // NW
CH = 128
NCH = PW // CH


PWH = (B * L) // NW
NCHH = PWH // CH
SPW = B // NW


def _sc_gather_body(table_hbm, idx_hbm, mask_hbm, out_h, out_s,
                    idx_v, rows_a, rows_b, big_i, mask_v, sums_v,
                    sem_a, sem_b, sem_i):
    wid = lax.axis_index("s") * NC + lax.axis_index("c")
    base = wid * PWH
    pltpu.sync_copy(idx_hbm.at[pl.ds(base, PWH)], idx_v.at[pl.ds(0, PWH)])
    pltpu.sync_copy(idx_hbm.at[pl.ds(B * L + base, PWH)],
                    idx_v.at[pl.ds(PWH, PWH)])
    pltpu.sync_copy(mask_hbm.at[pl.ds(wid * SPW, SPW)], mask_v)

    def item_copy(c):
        return pltpu.make_async_copy(
            table_hbm.at[idx_v.at[pl.ds(PWH + c * CH, CH)]],
            big_i.at[pl.ds(c * CH, CH)], sem_i)

    for c in range(NCHH):
        pltpu.async_copy(
            table_hbm.at[idx_v.at[pl.ds(PWH + c * CH, CH)]],
            big_i.at[pl.ds(c * CH, CH)], sem_i)

    bufs = (rows_a, rows_b)
    sems = (sem_a, sem_b)

    def start(c):
        pltpu.async_copy(
            table_hbm.at[idx_v.at[pl.ds(c * CH, CH)]], bufs[c % 2],
            sems[c % 2])

    start(0)
    for c in range(NCHH):
        if c + 1 < NCHH:
            start(c + 1)
        pltpu.make_async_copy(
            table_hbm.at[idx_v.at[pl.ds(c * CH, CH)]], bufs[c % 2],
            sems[c % 2]).wait()
        pltpu.sync_copy(bufs[c % 2], out_h.at[pl.ds(base + c * CH, CH)])

    for c in range(NCHH):
        item_copy(c).wait()

    def session(j, _):
        ma = mask_v[j, pl.ds(0, 16)]
        mb = mask_v[j, pl.ds(16, 16)]
        accs = [jnp.zeros((16,), jnp.float32) for _ in range(DIM // 16)]
        for r in range(L):
            mv = ma[r] if r < 16 else mb[r - 16]
            for cc in range(DIM // 16):
                accs[cc] = accs[cc] + mv * big_i[j * L + r,
                                                 pl.ds(cc * 16, 16)]
        for cc in range(DIM // 16):
            sums_v[j, pl.ds(cc * 16, 16)] = accs[cc]
        return 0

    jax.lax.fori_loop(0, SPW, session, 0)
    pltpu.sync_copy(sums_v, out_s.at[pl.ds(wid * SPW, SPW)])


def _sc_gather(embedding, flat_idx, mask_item):
    mesh = plsc.VectorSubcoreMesh(core_axis_name="c", subcore_axis_name="s")
    return pl.kernel(
        _sc_gather_body,
        mesh=mesh,
        out_type=[
            jax.ShapeDtypeStruct((B * L, DIM), jnp.float32),
            jax.ShapeDtypeStruct((B, DIM), jnp.float32),
        ],
        scratch_types=[
            pltpu.VMEM((PW,), jnp.int32),
            pltpu.VMEM((CH, DIM), jnp.float32),
            pltpu.VMEM((CH, DIM), jnp.float32),
            pltpu.VMEM((PWH, DIM), jnp.float32),
            pltpu.VMEM((SPW, 32), jnp.float32),
            pltpu.VMEM((SPW, DIM), jnp.float32),
            pltpu.SemaphoreType.DMA,
            pltpu.SemaphoreType.DMA,
            pltpu.SemaphoreType.DMA,
        ],
    )(embedding, flat_idx, mask_item)


def _leaky(x):
    return jnp.where(x >= 0, x, 0.2 * x)


def _tc_body(h_ref, adj_ref, amat_ref, sums_ref, maskf_ref,
             w1_ref, b1_ref, w2_ref, b2_ref, out_ref, anchor_ref):
    hload = h_ref[...]
    amat = amat_ref[...]
    adjload = adj_ref[...]
    rg = jax.lax.broadcasted_iota(jnp.int32, (SBL, SBL), 0) % SB
    cg = jax.lax.broadcasted_iota(jnp.int32, (SBL, SBL), 1) % SB
    bd = rg == cg
    selc = jax.lax.broadcasted_iota(jnp.int32, (L, SBL), 0)
    selj = jax.lax.broadcasted_iota(jnp.int32, (L, SBL), 1) // SB
    sel = (selc == selj).astype(jnp.float32)

    hss, e_alls, mrs = [], [], []
    for s in range(NSUB):
        hs = hload[:, s * SB:(s + 1) * SB, :].reshape(SBL, DIM)
        u = jnp.concatenate(
            [hs * amat[k:k + 1, :] for k in range(4)], axis=0)
        e_all = jax.lax.dot_general(
            u, hs, (((1,), (1,)), ((), ())),
            preferred_element_type=jnp.float32)
        adjs = adjload[:, s * SB:(s + 1) * SB, :].reshape(SBL, L)
        rep = jax.lax.dot_general(
            adjs.astype(jnp.float32), sel, (((1,), (0,)), ((), ())),
            preferred_element_type=jnp.float32)
        hss.append(hs)
        e_alls.append(e_all)
        mrs.append(jnp.where(bd, rep, 5.0))

    for s in range(NSUB):
        e_all, mr = e_alls[s], mrs[s]
        alpha = jnp.where(
            mr == 1.0, e_all[0:SBL, :],
            jnp.where(mr == 2.0, e_all[SBL:2 * SBL, :],
                      jnp.where(mr == 3.0, e_all[2 * SBL:3 * SBL, :],
                                jnp.where(mr == 4.0, e_all[3 * SBL:, :],
                                          jnp.where(mr == 5.0, -jnp.inf,
                                                    _NEG)))))
        alpha = _leaky(alpha)
        m = jnp.max(alpha, axis=1, keepdims=True)
        p = jnp.exp(alpha - m)
        p = p / jnp.sum(p, axis=1, keepdims=True)
        out_s = jax.lax.dot_general(
            p, hss[s], (((1,), (0,)), ((), ())),
            preferred_element_type=jnp.float32)
        out_ref[:, s * SB:(s + 1) * SB, :] = out_s.reshape(L, SB, DIM)

    maskf = maskf_ref[...].astype(jnp.float32)
    s_emb = sums_ref[...]
    cnt = jnp.sum(maskf, axis=1, keepdims=True)
    mean = s_emb / cnt
    hidden = jnp.maximum(
        jax.lax.dot_general(mean, w1_ref[...], (((1,), (0,)), ((), ())),
                            preferred_element_type=jnp.float32)
        + b1_ref[0:1, :], 0.0)
    anchor_ref[...] = (
        jax.lax.dot_general(hidden, w2_ref[...], (((1,), (0,)), ((), ())),
                            preferred_element_type=jnp.float32)
        + b2_ref[0:1, :])


def _tc_call(ht, item_sums, adj_t, amat, mask_item, w1, b1, w2, b2):
    grid = (B // BB,)
    return pl.pallas_call(
        _tc_body,
        grid=grid,
        in_specs=[
            pl.BlockSpec((L, BB, DIM), lambda i: (0, i, 0)),
            pl.BlockSpec((L, BB, L), lambda i: (0, i, 0)),
            pl.BlockSpec((8, DIM), lambda i: (0, 0)),
            pl.BlockSpec((BB, DIM), lambda i: (i, 0)),
            pl.BlockSpec((BB, L), lambda i: (i, 0)),
            pl.BlockSpec((DIM, DIM), lambda i: (0, 0)),
            pl.BlockSpec((8, DIM), lambda i: (0, 0)),
            pl.BlockSpec((DIM, DIM), lambda i: (0, 0)),
            pl.BlockSpec((8, DIM), lambda i: (0, 0)),
        ],
        out_specs=[
            pl.BlockSpec((L, BB, DIM), lambda i: (0, i, 0)),
            pl.BlockSpec((BB, DIM), lambda i: (i, 0)),
        ],
        out_shape=[
            jax.ShapeDtypeStruct((L, B, DIM), jnp.float32),
            jax.ShapeDtypeStruct((B, DIM), jnp.float32),
        ],
        interpret=_INTERPRET,
    )(ht, adj_t, amat, item_sums, mask_item, w1, b1, w2, b2)


def kernel(inputs, adj, mask_item, item, data, hg_adj, embedding, adj_all,
           num, a_0, a_1, a_2, a_3, mlp_w1, mlp_b1, mlp_w2, mlp_b2):
    flat_idx = jnp.concatenate([inputs.T.reshape(-1), item.reshape(-1)])
    maskp = jnp.pad(mask_item.astype(jnp.float32), ((0, 0), (0, 32 - L)))
    h_flat, item_sums = _sc_gather(embedding, flat_idx, maskp)
    ht = h_flat.reshape(L, B, DIM)
    adj_t = adj.transpose(1, 0, 2)
    amat = jnp.concatenate(
        [a_0.T, a_1.T, a_2.T, a_3.T,
         jnp.zeros((4, DIM), jnp.float32)], axis=0)
    b1 = jnp.broadcast_to(mlp_b1[None, :], (8, DIM))
    b2 = jnp.broadcast_to(mlp_b2[None, :], (8, DIM))
    out_t, anchor = _tc_call(ht, item_sums, adj_t, amat, mask_item,
                             mlp_w1, b1, mlp_w2, b2)
    return (out_t.transpose(1, 0, 2), anchor)


# speedup vs baseline: 3.3487x; 1.0109x over previous
import functools

import jax
import jax.numpy as jnp
from jax import lax
from jax.experimental import pallas as pl
from jax.experimental.pallas import tpu as pltpu
from jax.experimental.pallas import tpu_sc as plsc

DIM = 128
L = 20
B = 1024
SB = 8
SBL = SB * L
BB = 256
NSUB = BB // SB

_NEG = -9e15

_INTERPRET = False

NC = 2
NS = 16
NW = NC * NS
TOT = 2 * B * L
PW = TOT // NW
CH = 128
NCH = PW // CH


PWH = (B * L) // NW
NCHH = PWH // CH
SPW = B // NW


def _sc_gather_body(table_hbm, idx_hbm, mask_hbm, out_h, out_s,
                    idx_v, rows_a, rows_b, big_i, mask_v, sums_v,
                    sem_a, sem_b, sem_i):
    wid = lax.axis_index("s") * NC + lax.axis_index("c")
    base = wid * PWH
    pltpu.sync_copy(idx_hbm.at[pl.ds(base, PWH)], idx_v.at[pl.ds(0, PWH)])
    pltpu.sync_copy(idx_hbm.at[pl.ds(B * L + base, PWH)],
                    idx_v.at[pl.ds(PWH, PWH)])
    pltpu.sync_copy(mask_hbm.at[pl.ds(wid * SPW, SPW)], mask_v)

    def item_copy(c):
        return pltpu.make_async_copy(
            table_hbm.at[idx_v.at[pl.ds(PWH + c * CH, CH)]],
            big_i.at[pl.ds(c * CH, CH)], sem_i)

    for c in range(NCHH):
        pltpu.async_copy(
            table_hbm.at[idx_v.at[pl.ds(PWH + c * CH, CH)]],
            big_i.at[pl.ds(c * CH, CH)], sem_i)

    bufs = (rows_a, rows_b)
    sems = (sem_a, sem_b)

    def start(c):
        pltpu.async_copy(
            table_hbm.at[idx_v.at[pl.ds(c * CH, CH)]], bufs[c % 2],
            sems[c % 2])

    start(0)
    for c in range(NCHH):
        if c + 1 < NCHH:
            start(c + 1)
        pltpu.make_async_copy(
            table_hbm.at[idx_v.at[pl.ds(c * CH, CH)]], bufs[c % 2],
            sems[c % 2]).wait()
        pltpu.sync_copy(bufs[c % 2], out_h.at[pl.ds(base + c * CH, CH)])

    for c in range(NCHH):
        item_copy(c).wait()

    def session(j, _):
        ma = mask_v[j, pl.ds(0, 16)]
        mb = mask_v[j, pl.ds(16, 16)]
        accs = [jnp.zeros((16,), jnp.float32) for _ in range(DIM // 16)]
        for r in range(L):
            mv = ma[r] if r < 16 else mb[r - 16]
            for cc in range(DIM // 16):
                accs[cc] = accs[cc] + mv * big_i[j * L + r,
                                                 pl.ds(cc * 16, 16)]
        for cc in range(DIM // 16):
            sums_v[j, pl.ds(cc * 16, 16)] = accs[cc]
        return 0

    jax.lax.fori_loop(0, SPW, session, 0)
    pltpu.sync_copy(sums_v, out_s.at[pl.ds(wid * SPW, SPW)])


def _sc_gather(embedding, flat_idx, mask_item):
    mesh = plsc.VectorSubcoreMesh(core_axis_name="c", subcore_axis_name="s")
    return pl.kernel(
        _sc_gather_body,
        mesh=mesh,
        out_type=[
            jax.ShapeDtypeStruct((B * L, DIM), jnp.float32),
            jax.ShapeDtypeStruct((B, DIM), jnp.float32),
        ],
        scratch_types=[
            pltpu.VMEM((PW,), jnp.int32),
            pltpu.VMEM((CH, DIM), jnp.float32),
            pltpu.VMEM((CH, DIM), jnp.float32),
            pltpu.VMEM((PWH, DIM), jnp.float32),
            pltpu.VMEM((SPW, 32), jnp.float32),
            pltpu.VMEM((SPW, DIM), jnp.float32),
            pltpu.SemaphoreType.DMA,
            pltpu.SemaphoreType.DMA,
            pltpu.SemaphoreType.DMA,
        ],
    )(embedding, flat_idx, mask_item)


def _leaky(x):
    return jnp.where(x >= 0, x, 0.2 * x)


def _tc_body(h_ref, adj_ref, amat_ref, sums_ref, maskf_ref,
             w1_ref, b1_ref, w2_ref, b2_ref, out_ref, anchor_ref):
    hload = h_ref[...]
    amat = amat_ref[...]
    adjload = adj_ref[...]
    rg = jax.lax.broadcasted_iota(jnp.int32, (SBL, SBL), 0) % SB
    cg = jax.lax.broadcasted_iota(jnp.int32, (SBL, SBL), 1) % SB
    bd = rg == cg
    selc = jax.lax.broadcasted_iota(jnp.int32, (L, SBL), 0)
    selj = jax.lax.broadcasted_iota(jnp.int32, (L, SBL), 1) // SB
    sel = (selc == selj).astype(jnp.float32)

    hss, e_alls, mrs = [], [], []
    for s in range(NSUB):
        hs = hload[:, s * SB:(s + 1) * SB, :].reshape(SBL, DIM)
        u = jnp.concatenate(
            [hs * amat[k:k + 1, :] for k in range(4)], axis=0)
        e_all = jax.lax.dot_general(
            u, hs, (((1,), (1,)), ((), ())),
            preferred_element_type=jnp.float32)
        adjs = adjload[:, s * SB:(s + 1) * SB, :].reshape(SBL, L)
        rep = jax.lax.dot_general(
            adjs.astype(jnp.float32), sel, (((1,), (0,)), ((), ())),
            preferred_element_type=jnp.float32)
        hss.append(hs)
        e_alls.append(e_all)
        mrs.append(jnp.where(bd, rep, 5.0))

    for s in range(NSUB):
        e_all, mr = e_alls[s], mrs[s]
        alpha = jnp.where(
            mr == 1.0, e_all[0:SBL, :],
            jnp.where(mr == 2.0, e_all[SBL:2 * SBL, :],
                      jnp.where(mr == 3.0, e_all[2 * SBL:3 * SBL, :],
                                jnp.where(mr == 4.0, e_all[3 * SBL:, :],
                                          jnp.where(mr == 5.0, -jnp.inf,
                                                    _NEG)))))
        alpha = _leaky(alpha)
        m = jnp.max(alpha, axis=1, keepdims=True)
        p = jnp.exp(alpha - m)
        p = p / jnp.sum(p, axis=1, keepdims=True)
        out_s = jax.lax.dot_general(
            p, hss[s], (((1,), (0,)), ((), ())),
            preferred_element_type=jnp.float32)
        out_ref[:, s * SB:(s + 1) * SB, :] = out_s.reshape(L, SB, DIM)

    maskf = maskf_ref[...].astype(jnp.float32)
    s_emb = sums_ref[...]
    cnt = jnp.sum(maskf, axis=1, keepdims=True)
    mean = s_emb / cnt
    hidden = jnp.maximum(
        jax.lax.dot_general(mean, w1_ref[...], (((1,), (0,)), ((), ())),
                            preferred_element_type=jnp.float32)
        + b1_ref[0:1, :], 0.0)
    anchor_ref[...] = (
        jax.lax.dot_general(hidden, w2_ref[...], (((1,), (0,)), ((), ())),
                            preferred_element_type=jnp.float32)
        + b2_ref[0:1, :])


def _tc_call(ht, item_sums, adj_t, amat, mask_item, w1, b1, w2, b2):
    grid = (B // BB,)
    return pl.pallas_call(
        _tc_body,
        grid=grid,
        in_specs=[
            pl.BlockSpec((L, BB, DIM), lambda i: (0, i, 0)),
            pl.BlockSpec((L, BB, L), lambda i: (0, i, 0)),
            pl.BlockSpec((8, DIM), lambda i: (0, 0)),
            pl.BlockSpec((BB, DIM), lambda i: (i, 0)),
            pl.BlockSpec((BB, L), lambda i: (i, 0)),
            pl.BlockSpec((DIM, DIM), lambda i: (0, 0)),
            pl.BlockSpec((8, DIM), lambda i: (0, 0)),
            pl.BlockSpec((DIM, DIM), lambda i: (0, 0)),
            pl.BlockSpec((8, DIM), lambda i: (0, 0)),
        ],
        out_specs=[
            pl.BlockSpec((L, BB, DIM), lambda i: (0, i, 0)),
            pl.BlockSpec((BB, DIM), lambda i: (i, 0)),
        ],
        out_shape=[
            jax.ShapeDtypeStruct((L, B, DIM), jnp.float32),
            jax.ShapeDtypeStruct((B, DIM), jnp.float32),
        ],
        interpret=_INTERPRET,
    )(ht, adj_t, amat, item_sums, mask_item, w1, b1, w2, b2)


def kernel(inputs, adj, mask_item, item, data, hg_adj, embedding, adj_all,
           num, a_0, a_1, a_2, a_3, mlp_w1, mlp_b1, mlp_w2, mlp_b2):
    flat_idx = jnp.concatenate([inputs.T.reshape(-1), item.reshape(-1)])
    maskp = jnp.pad(mask_item.astype(jnp.float32), ((0, 0), (0, 32 - L)))
    h_flat, item_sums = _sc_gather(embedding, flat_idx, maskp)
    ht = h_flat.reshape(L, B, DIM)
    adj_t = adj.transpose(1, 0, 2)
    amat = jnp.concatenate(
        [a_0.T, a_1.T, a_2.T, a_3.T,
         jnp.zeros((4, DIM), jnp.float32)], axis=0)
    b1 = jnp.broadcast_to(mlp_b1[None, :], (8, DIM))
    b2 = jnp.broadcast_to(mlp_b2[None, :], (8, DIM))
    out_t, anchor = _tc_call(ht, item_sums, adj_t, amat, mask_item,
                             mlp_w1, b1, mlp_w2, b2)
    return (out_t.transpose(1, 0, 2), anchor)
